# baseline (speedup 1.0000x reference)
import functools

import jax
import jax.numpy as jnp
from jax import lax
from jax.experimental import pallas as pl
from jax.experimental.pallas import tpu as pltpu
from jax.experimental.pallas import tpu_sc as plsc

EPS = 1e-5

NC = 2
NS = 16
NW = NC * NS

CHUNK = 128
NBUF = 8


def _sc_gather(ids3d, table, n, d):
    per_w = n // NW
    n_chunks = per_w // CHUNK
    n_groups = n_chunks // NBUF
    mesh = plsc.VectorSubcoreMesh(
        core_axis_name="c", subcore_axis_name="s", num_cores=NC, num_subcores=NS
    )

    @functools.partial(
        pl.kernel,
        out_type=jax.ShapeDtypeStruct((n, d), jnp.float32),
        mesh=mesh,
        scratch_types=[
            pltpu.VMEM((n_chunks, CHUNK), jnp.int32),
            pltpu.VMEM((NBUF, CHUNK, d), jnp.float32),
            pltpu.SemaphoreType.DMA((NBUF,)),
        ],
        compiler_params=pltpu.CompilerParams(use_tc_tiling_on_sc=False),
    )
    def gather_kernel(ids_hbm, table_hbm, out_hbm, idx_v, rows_v, sems):
        wid = lax.axis_index("s") * NC + lax.axis_index("c")
        base = wid * per_w
        pltpu.sync_copy(ids_hbm.at[wid], idx_v)

        def start(j, buf):
            pltpu.async_copy(table_hbm.at[idx_v.at[j]], rows_v.at[buf], sems.at[buf])

        def finish(j, buf):
            pltpu.make_async_copy(
                table_hbm.at[idx_v.at[j]], rows_v.at[buf], sems.at[buf]
            ).wait()
            pltpu.sync_copy(rows_v.at[buf], out_hbm.at[pl.ds(base + j * CHUNK, CHUNK)])

        for buf in range(NBUF):
            start(buf, buf)

        def group(g, carry):
            for buf in range(NBUF):
                j = g * NBUF + buf
                finish(j, buf)
                start(j + NBUF, buf)
            return carry

        lax.fori_loop(0, n_groups - 1, group, 0, unroll=False)

        for buf in range(NBUF):
            finish((n_groups - 1) * NBUF + buf, buf)

    return gather_kernel(ids3d, table)


def _tc_dense(x, wt, b, gamma, beta, n, d, o, rows_per_blk):
    def body(x_ref, wt_ref, b_ref, g_ref, be_ref, o_ref):
        xv = x_ref[...]
        p = lax.dot_general(
            xv, wt_ref[...], (((1,), (0,)), ((), ())),
            preferred_element_type=jnp.float32,
        )
        p = p + b_ref[...]
        mean = jnp.mean(p, axis=1, keepdims=True)
        c = p - mean
        var = jnp.mean(c * c, axis=1, keepdims=True)
        o_ref[...] = c * lax.rsqrt(var + EPS) * g_ref[...] + be_ref[...]

    grid = (n // rows_per_blk,)
    return pl.pallas_call(
        body,
        grid=grid,
        in_specs=[
            pl.BlockSpec((rows_per_blk, d), lambda i: (i, 0)),
            pl.BlockSpec((d, o), lambda i: (0, 0)),
            pl.BlockSpec((1, o), lambda i: (0, 0)),
            pl.BlockSpec((1, o), lambda i: (0, 0)),
            pl.BlockSpec((1, o), lambda i: (0, 0)),
        ],
        out_specs=pl.BlockSpec((rows_per_blk, o), lambda i: (i, 0)),
        out_shape=jax.ShapeDtypeStruct((n, o), jnp.float32),
        compiler_params=pltpu.CompilerParams(
            dimension_semantics=("arbitrary",),
        ),
    )(x, wt, b, gamma, beta)


@jax.jit
def kernel(item_ids, feat_matrix, W, b, gamma, beta):
    B, L = item_ids.shape
    d = feat_matrix.shape[1]
    o = W.shape[0]
    n = B * L

    per_w = n // NW
    n_chunks = per_w // CHUNK
    ids3d = item_ids.reshape(NW, n_chunks, CHUNK)

    gathered = _sc_gather(ids3d, feat_matrix, n, d)

    out = _tc_dense(
        gathered,
        W.T,
        b.reshape(1, o),
        gamma.reshape(1, o),
        beta.reshape(1, o),
        n, d, o, rows_per_blk=2048,
    )
    return out.reshape(B, L, o)


# speedup vs baseline: 1.4254x; 1.4254x over previous
import functools

import jax
import jax.numpy as jnp
from jax import lax
from jax.experimental import pallas as pl
from jax.experimental.pallas import tpu as pltpu
from jax.experimental.pallas import tpu_sc as plsc

EPS = 1e-5

NC = 2
NS = 16
NW = NC * NS

CH_IDX = 128
CH_P = 64
NBUF = 8


def _sc_gather_packed(ids_pack, table, n, d):
    half = n // 2
    per_w = half // NW
    n_chunks = per_w // CH_P
    n_groups = n_chunks // NBUF
    mesh = plsc.VectorSubcoreMesh(
        core_axis_name="c", subcore_axis_name="s", num_cores=NC, num_subcores=NS
    )

    @functools.partial(
        pl.kernel,
        out_type=jax.ShapeDtypeStruct((half, 2 * d), jnp.float32),
        mesh=mesh,
        scratch_types=[
            pltpu.VMEM((n_chunks, CH_IDX), jnp.int32),
            pltpu.VMEM((NBUF, CH_IDX, d), jnp.float32),
            pltpu.SemaphoreType.DMA((NBUF,)),
        ],
        compiler_params=pltpu.CompilerParams(use_tc_tiling_on_sc=False),
    )
    def gather_kernel(ids_hbm, table_hbm, out_hbm, idx_v, rows_v, sems):
        wid = lax.axis_index("s") * NC + lax.axis_index("c")
        base = wid * per_w
        pltpu.sync_copy(ids_hbm.at[wid], idx_v)

        def start(j, buf):
            pltpu.async_copy(table_hbm.at[idx_v.at[j]], rows_v.at[buf], sems.at[buf])

        def finish(j, buf):
            pltpu.make_async_copy(
                table_hbm.at[idx_v.at[j]], rows_v.at[buf], sems.at[buf]
            ).wait()
            row0 = base + j * CH_P
            pltpu.sync_copy(
                rows_v.at[buf, pl.ds(0, CH_P)],
                out_hbm.at[pl.ds(row0, CH_P), pl.ds(0, d)],
            )
            pltpu.sync_copy(
                rows_v.at[buf, pl.ds(CH_P, CH_P)],
                out_hbm.at[pl.ds(row0, CH_P), pl.ds(d, d)],
            )

        for buf in range(NBUF):
            start(buf, buf)

        def group(g, carry):
            for buf in range(NBUF):
                j = g * NBUF + buf
                finish(j, buf)
                start(j + NBUF, buf)
            return carry

        lax.fori_loop(0, n_groups - 1, group, 0, unroll=False)

        for buf in range(NBUF):
            finish((n_groups - 1) * NBUF + buf, buf)

    return gather_kernel(ids_pack, table)


def _tc_dense(x_packed, wt2, b2, gamma2, beta2, n, d, o, rows_per_blk):
    half = n // 2
    h_blocks = half // rows_per_blk

    def body(x_ref, wt_ref, b_ref, g_ref, be_ref, o_ref):
        xv = x_ref[...]
        p2 = lax.dot_general(
            xv, wt_ref[...], (((1,), (0,)), ((), ())),
            preferred_element_type=jnp.float32,
        )
        p2 = p2 + b_ref[...]
        for hh in range(2):
            p = p2[:, hh * o:(hh + 1) * o]
            mean = jnp.mean(p, axis=1, keepdims=True)
            c = p - mean
            var = jnp.mean(c * c, axis=1, keepdims=True)
            o_ref[hh] = (
                c * lax.rsqrt(var + EPS) * g_ref[:, hh * o:(hh + 1) * o]
                + be_ref[:, hh * o:(hh + 1) * o]
            )

    grid = (h_blocks,)
    return pl.pallas_call(
        body,
        grid=grid,
        in_specs=[
            pl.BlockSpec((rows_per_blk, 2 * d), lambda i: (i, 0)),
            pl.BlockSpec((2 * d, 2 * o), lambda i: (0, 0)),
            pl.BlockSpec((1, 2 * o), lambda i: (0, 0)),
            pl.BlockSpec((1, 2 * o), lambda i: (0, 0)),
            pl.BlockSpec((1, 2 * o), lambda i: (0, 0)),
        ],
        out_specs=pl.BlockSpec((2, rows_per_blk, o), lambda i: (0, i, 0)),
        out_shape=jax.ShapeDtypeStruct((2, half, o), jnp.float32),
        compiler_params=pltpu.CompilerParams(
            dimension_semantics=("arbitrary",),
        ),
    )(x_packed, wt2, b2, gamma2, beta2)


@jax.jit
def kernel(item_ids, feat_matrix, W, b, gamma, beta):
    B, L = item_ids.shape
    d = feat_matrix.shape[1]
    o = W.shape[0]
    n = B * L
    half = n // 2

    per_w = half // NW
    n_chunks = per_w // CH_P
    ids_flat = item_ids.reshape(n)
    lo = ids_flat[:half].reshape(NW, n_chunks, 1, CH_P)
    hi = ids_flat[half:].reshape(NW, n_chunks, 1, CH_P)
    ids_pack = jnp.concatenate([lo, hi], axis=2).reshape(NW, n_chunks, CH_IDX)

    packed = _sc_gather_packed(ids_pack, feat_matrix, n, d)

    wt = W.T
    zeros = jnp.zeros((d, o), dtype=wt.dtype)
    wt2 = jnp.concatenate(
        [
            jnp.concatenate([wt, zeros], axis=1),
            jnp.concatenate([zeros, wt], axis=1),
        ],
        axis=0,
    )
    b2 = jnp.tile(b, 2).reshape(1, 2 * o)
    gamma2 = jnp.tile(gamma, 2).reshape(1, 2 * o)
    beta2 = jnp.tile(beta, 2).reshape(1, 2 * o)

    out = _tc_dense(packed, wt2, b2, gamma2, beta2, n, d, o, rows_per_blk=2048)
    return out.reshape(B, L, o)
